# Initial kernel scaffold; baseline (speedup 1.0000x reference)
import jax
import jax.numpy as jnp
from jax.experimental import pallas as pl


def kernel(input, table):
    raise NotImplementedError("write your pallas kernel here")


# speedup vs baseline: 1.0943x; 1.0943x over previous
import functools

import jax
import jax.numpy as jnp
from jax import lax
from jax.experimental import pallas as pl
from jax.experimental.pallas import tpu as pltpu
from jax.experimental.pallas import tpu_sc as plsc

EMB_DIM = 32
NUM_CORES = 2
NUM_SUBCORES = 16
NUM_WORKERS = NUM_CORES * NUM_SUBCORES

CHUNK = 1024


@functools.partial(jax.jit, static_argnames=())
def _gather_flat(idx, table):
    (B,) = idx.shape
    b_per_w = B // NUM_WORKERS
    n_chunks = b_per_w // CHUNK

    mesh = plsc.VectorSubcoreMesh(core_axis_name="c", subcore_axis_name="s")

    @functools.partial(
        pl.kernel,
        mesh=mesh,
        out_type=jax.ShapeDtypeStruct((B, EMB_DIM), jnp.float32),
        scratch_types=[
            pltpu.VMEM((CHUNK,), jnp.int32),
            pltpu.VMEM((CHUNK, EMB_DIM), jnp.float32),
            pltpu.SemaphoreType.DMA,
        ],
        compiler_params=pltpu.CompilerParams(use_tc_tiling_on_sc=False),
    )
    def k(table_hbm, idx_hbm, out_hbm, idx_v, rows_v, sem):
        wid = lax.axis_index("s") * NUM_CORES + lax.axis_index("c")
        base = wid * b_per_w

        def body(i, carry):
            off = base + i * CHUNK
            pltpu.sync_copy(idx_hbm.at[pl.ds(off, CHUNK)], idx_v)
            pltpu.async_copy(table_hbm.at[idx_v], rows_v, sem).wait()
            pltpu.sync_copy(rows_v, out_hbm.at[pl.ds(off, CHUNK)])
            return carry

        lax.fori_loop(0, n_chunks, body, 0)

    return k(table, idx)


def kernel(input, table):
    B0, B1 = input.shape
    idx = input.reshape(B0 * B1).astype(jnp.int32)
    out = _gather_flat(idx, table)
    return out.reshape(B0, B1, EMB_DIM)


# speedup vs baseline: 1.1136x; 1.0176x over previous
import functools

import jax
import jax.numpy as jnp
from jax import lax
from jax.experimental import pallas as pl
from jax.experimental.pallas import tpu as pltpu
from jax.experimental.pallas import tpu_sc as plsc

EMB_DIM = 32
NUM_CORES = 2
NUM_SUBCORES = 16
NUM_WORKERS = NUM_CORES * NUM_SUBCORES

CHUNK = 640
NBUF = 4


@jax.jit
def _gather_flat(idx, table):
    (B,) = idx.shape
    b_per_w = B // NUM_WORKERS
    n_chunks = b_per_w // CHUNK
    assert n_chunks % NBUF == 0

    mesh = plsc.VectorSubcoreMesh(core_axis_name="c", subcore_axis_name="s")

    @functools.partial(
        pl.kernel,
        mesh=mesh,
        out_type=jax.ShapeDtypeStruct((B, EMB_DIM), jnp.float32),
        scratch_types=[
            pltpu.VMEM((b_per_w,), jnp.int32),
            pltpu.VMEM((NBUF, CHUNK, EMB_DIM), jnp.float32),
            pltpu.SemaphoreType.DMA((NBUF,)),
            pltpu.SemaphoreType.DMA((NBUF,)),
        ],
        compiler_params=pltpu.CompilerParams(use_tc_tiling_on_sc=False),
    )
    def k(table_hbm, idx_hbm, out_hbm, idx_all, rows_v, sem_g, sem_s):
        wid = lax.axis_index("s") * NUM_CORES + lax.axis_index("c")
        base = wid * b_per_w
        pltpu.sync_copy(idx_hbm.at[pl.ds(base, b_per_w)], idx_all)

        def gather_desc(g, slot):
            return pltpu.make_async_copy(
                table_hbm.at[idx_all.at[pl.ds(g * CHUNK, CHUNK)]],
                rows_v.at[slot],
                sem_g.at[slot],
            )

        def store_desc(g, slot):
            return pltpu.make_async_copy(
                rows_v.at[slot],
                out_hbm.at[pl.ds(base + g * CHUNK, CHUNK)],
                sem_s.at[slot],
            )

        gather_desc(0, 0).start()

        @pl.loop(0, n_chunks, step=NBUF)
        def block(g0):
            for j in range(NBUF):
                g = g0 + j
                nxt = g + 1
                s_cur = j
                s_nxt = (j + 1) % NBUF

                @pl.when(nxt < n_chunks)
                def _():
                    @pl.when(nxt >= NBUF)
                    def _():
                        store_desc(nxt - NBUF, s_nxt).wait()

                    gather_desc(nxt, s_nxt).start()

                gather_desc(g, s_cur).wait()
                store_desc(g, s_cur).start()

        for j in range(NBUF):
            store_desc(n_chunks - NBUF + j, j).wait()

    return k(table, idx)


def kernel(input, table):
    B0, B1 = input.shape
    idx = input.reshape(B0 * B1).astype(jnp.int32)
    out = _gather_flat(idx, table)
    return out.reshape(B0, B1, EMB_DIM)
